# baseline (speedup 1.0000x reference)
import functools

import jax
import jax.numpy as jnp
from jax import lax
from jax.experimental import pallas as pl
from jax.experimental.pallas import tpu as pltpu
from jax.experimental.pallas import tpu_sc as plsc


@functools.lru_cache(maxsize=None)
def _build_sc_kernel(B, NCLS, C, H, W):
    D = C * H * W
    info = plsc.get_sparse_core_info()
    NC, NS, L = info.num_cores, info.num_subcores, info.num_lanes
    NW = NC * NS
    CH = C * H
    CHPW = CH // NW
    WT = 8
    BC = 1024
    NBUF = 4
    NB = B // BC
    NWT = W // WT
    NCHUNK = CHPW * NWT * NB
    U = 8

    mesh = plsc.VectorSubcoreMesh(core_axis_name="c", subcore_axis_name="s")

    @functools.partial(
        pl.kernel,
        mesh=mesh,
        out_type=jax.ShapeDtypeStruct((C, H, W, B), jnp.float32),
        compiler_params=pltpu.CompilerParams(needs_layout_passes=False),
        scratch_types=[
            pltpu.VMEM((B,), jnp.int32),
            pltpu.VMEM((NCLS * (D + 1),), jnp.float32),
            pltpu.VMEM((NCLS * (D + 1),), jnp.float32),
        ] + [pltpu.VMEM((WT, BC), jnp.float32) for _ in range(NBUF)]
          + [pltpu.SemaphoreType.DMA for _ in range(2 * NBUF)],
    )
    def sc_fma(lab_hbm, mean_hbm, std_hbm, noise_hbm, out_hbm,
               idx_v, mtab, stab, nbuf0, nbuf1, nbuf2, nbuf3,
               in_sem0, in_sem1, in_sem2, in_sem3,
               out_sem0, out_sem1, out_sem2, out_sem3):
        nv = noise_hbm.reshape(CH, W, B)
        ov = out_hbm.reshape(CH, W, B)
        wid = lax.axis_index("s") * NC + lax.axis_index("c")
        base_ch = wid * CHPW
        nbufs = (nbuf0, nbuf1, nbuf2, nbuf3)
        in_sems = (in_sem0, in_sem1, in_sem2, in_sem3)
        out_sems = (out_sem0, out_sem1, out_sem2, out_sem3)

        pltpu.sync_copy(lab_hbm, idx_v)
        h_m = pltpu.async_copy(mean_hbm, mtab, out_sem0)
        h_s = pltpu.async_copy(std_hbm, stab, out_sem1)

        def coords(c):
            chrow = base_ch + c // (NB * NWT)
            w0 = ((c // NB) % NWT) * WT
            b0 = (c % NB) * BC
            return chrow, w0, b0

        def issue_in(c, p):
            chrow, w0, b0 = coords(c)
            pltpu.async_copy(nv.at[chrow, pl.ds(w0, WT), pl.ds(b0, BC)],
                             nbufs[p], in_sems[p])

        def wait_in(p):
            pltpu.make_async_copy(nv.at[0, pl.ds(0, WT), pl.ds(0, BC)],
                                  nbufs[p], in_sems[p]).wait()

        def issue_out(c, p):
            chrow, w0, b0 = coords(c)
            pltpu.async_copy(nbufs[p], ov.at[chrow, pl.ds(w0, WT), pl.ds(b0, BC)],
                             out_sems[p])

        def wait_out(p):
            pltpu.make_async_copy(nbufs[p], ov.at[0, pl.ds(0, WT), pl.ds(0, BC)],
                                  out_sems[p]).wait()

        def compute(c, p):
            nb = nbufs[p]
            chrow, w0, b0 = coords(c)
            dbase = chrow * W + w0
            dvecs = [jnp.full((L,), 1, jnp.int32) * (dbase + wi)
                     for wi in range(WT)]

            @plsc.parallel_loop(0, BC // L, step=1, unroll=U)
            def win_body(t):
                labv = idx_v[pl.ds(b0 + t * L, L)]
                sl = pl.ds(t * L, L)
                for wi in range(WT):
                    addr = labv + dvecs[wi]
                    n = nb[wi, sl]
                    m = plsc.load_gather(mtab, [addr])
                    s = plsc.load_gather(stab, [addr])
                    nb[wi, sl] = m + s * n

        issue_in(0, 0)
        issue_in(1, 1)
        h_m.wait()
        h_s.wait()

        for c in (0, 1):
            issue_in(c + 2, c + 2)
            wait_in(c)
            compute(c, c)
            issue_out(c, c)

        def quad(i, _):
            for j in range(4):
                c = 4 * i + 2 + j
                p = (2 + j) % NBUF
                q = j % NBUF
                wait_out(q)
                issue_in(c + 2, q)
                wait_in(p)
                compute(c, p)
                issue_out(c, p)
            return 0

        lax.fori_loop(0, (NCHUNK - 4) // 4, quad, 0)

        for c in (NCHUNK - 2, NCHUNK - 1):
            p = c % NBUF
            wait_in(p)
            compute(c, p)
            issue_out(c, p)

        for p in range(NBUF):
            wait_out(p)

    return sc_fma


def kernel(labels, class_means, class_stds, noise):
    B = labels.shape[0]
    NCLS, C, H, W = class_means.shape
    D = C * H * W
    sc_fma = _build_sc_kernel(B, NCLS, C, H, W)
    pad_mean = jnp.pad(class_means.reshape(NCLS, D), ((0, 0), (0, 1)))
    pad_std = jnp.pad(class_stds.reshape(NCLS, D), ((0, 0), (0, 1)))
    out_t = sc_fma(
        labels.astype(jnp.int32) * (D + 1),
        pad_mean.reshape(NCLS * (D + 1)),
        pad_std.reshape(NCLS * (D + 1)),
        jnp.transpose(noise, (1, 2, 3, 0)),
    )
    return jnp.transpose(out_t, (3, 0, 1, 2))


# speedup vs baseline: 1.2240x; 1.2240x over previous
import functools

import jax
import jax.numpy as jnp
from jax import lax
from jax.experimental import pallas as pl
from jax.experimental.pallas import tpu as pltpu
from jax.experimental.pallas import tpu_sc as plsc


@functools.lru_cache(maxsize=None)
def _build_sc_kernel(B, NCLS, C, H, W):
    D = C * H * W
    info = plsc.get_sparse_core_info()
    NC, NS, L = info.num_cores, info.num_subcores, info.num_lanes
    NW = NC * NS
    CH = C * H
    CHPW = CH // NW
    WT = 8
    BC = 1024
    NBUF = 4
    NB = B // BC
    NWT = W // WT
    NCHUNK = CHPW * NWT * NB
    U = 2

    mesh = plsc.VectorSubcoreMesh(core_axis_name="c", subcore_axis_name="s")

    @functools.partial(
        pl.kernel,
        mesh=mesh,
        out_type=jax.ShapeDtypeStruct((C, H, W, B), jnp.float32),
        compiler_params=pltpu.CompilerParams(needs_layout_passes=False),
        scratch_types=[
            pltpu.VMEM((B,), jnp.int32),
            pltpu.VMEM((NCLS * (D + 1),), jnp.float32),
            pltpu.VMEM((NCLS * (D + 1),), jnp.float32),
        ] + [pltpu.VMEM((WT, BC), jnp.float32) for _ in range(NBUF)]
          + [pltpu.SemaphoreType.DMA for _ in range(2 * NBUF)],
    )
    def sc_fma(lab_hbm, mean_hbm, std_hbm, noise_hbm, out_hbm,
               idx_v, mtab, stab, nbuf0, nbuf1, nbuf2, nbuf3,
               in_sem0, in_sem1, in_sem2, in_sem3,
               out_sem0, out_sem1, out_sem2, out_sem3):
        nv = noise_hbm.reshape(CH, W, B)
        ov = out_hbm.reshape(CH, W, B)
        wid = lax.axis_index("s") * NC + lax.axis_index("c")
        base_ch = wid * CHPW
        nbufs = (nbuf0, nbuf1, nbuf2, nbuf3)
        in_sems = (in_sem0, in_sem1, in_sem2, in_sem3)
        out_sems = (out_sem0, out_sem1, out_sem2, out_sem3)

        pltpu.sync_copy(lab_hbm, idx_v)
        h_m = pltpu.async_copy(mean_hbm, mtab, out_sem0)
        h_s = pltpu.async_copy(std_hbm, stab, out_sem1)

        def coords(c):
            chrow = base_ch + c // (NB * NWT)
            w0 = ((c // NB) % NWT) * WT
            b0 = (c % NB) * BC
            return chrow, w0, b0

        def issue_in(c, p):
            chrow, w0, b0 = coords(c)
            pltpu.async_copy(nv.at[chrow, pl.ds(w0, WT), pl.ds(b0, BC)],
                             nbufs[p], in_sems[p])

        def wait_in(p):
            pltpu.make_async_copy(nv.at[0, pl.ds(0, WT), pl.ds(0, BC)],
                                  nbufs[p], in_sems[p]).wait()

        def issue_out(c, p):
            chrow, w0, b0 = coords(c)
            pltpu.async_copy(nbufs[p], ov.at[chrow, pl.ds(w0, WT), pl.ds(b0, BC)],
                             out_sems[p])

        def wait_out(p):
            pltpu.make_async_copy(nbufs[p], ov.at[0, pl.ds(0, WT), pl.ds(0, BC)],
                                  out_sems[p]).wait()

        def compute(c, p):
            nb = nbufs[p]
            chrow, w0, b0 = coords(c)
            dbase = chrow * W + w0
            dvecs = [jnp.full((L,), 1, jnp.int32) * (dbase + wi)
                     for wi in range(WT)]

            @plsc.parallel_loop(0, BC // L, step=1, unroll=U)
            def win_body(t):
                labv = idx_v[pl.ds(b0 + t * L, L)]
                sl = pl.ds(t * L, L)
                for wi in range(WT):
                    addr = labv + dvecs[wi]
                    n = nb[wi, sl]
                    m = plsc.load_gather(mtab, [addr])
                    s = plsc.load_gather(stab, [addr])
                    nb[wi, sl] = m + s * n

        issue_in(0, 0)
        issue_in(1, 1)
        h_m.wait()
        h_s.wait()

        for c in (0, 1):
            issue_in(c + 2, c + 2)
            wait_in(c)
            compute(c, c)
            issue_out(c, c)

        def quad(i, _):
            for j in range(4):
                c = 4 * i + 2 + j
                p = (2 + j) % NBUF
                q = j % NBUF
                wait_out(q)
                issue_in(c + 2, q)
                wait_in(p)
                compute(c, p)
                issue_out(c, p)
            return 0

        lax.fori_loop(0, (NCHUNK - 4) // 4, quad, 0)

        for c in (NCHUNK - 2, NCHUNK - 1):
            p = c % NBUF
            wait_in(p)
            compute(c, p)
            issue_out(c, p)

        for p in range(NBUF):
            wait_out(p)

    return sc_fma


def kernel(labels, class_means, class_stds, noise):
    B = labels.shape[0]
    NCLS, C, H, W = class_means.shape
    D = C * H * W
    sc_fma = _build_sc_kernel(B, NCLS, C, H, W)
    pad_mean = jnp.pad(class_means.reshape(NCLS, D), ((0, 0), (0, 1)))
    pad_std = jnp.pad(class_stds.reshape(NCLS, D), ((0, 0), (0, 1)))
    out_t = sc_fma(
        labels.astype(jnp.int32) * (D + 1),
        pad_mean.reshape(NCLS * (D + 1)),
        pad_std.reshape(NCLS * (D + 1)),
        jnp.transpose(noise, (1, 2, 3, 0)),
    )
    return jnp.transpose(out_t, (3, 0, 1, 2))
